# baseline (speedup 1.0000x reference)
import jax
import jax.numpy as jnp
from jax import lax
from jax.experimental import pallas as pl
from jax.experimental.pallas import tpu as pltpu
from jax.experimental.pallas import tpu_sc as plsc

N_NODES = 100000
K = 32
D = 128
B = 16384

NC = 2
NS = 16
NW = NC * NS
BPW = B // NW
C = 8
NCH = BPW // C
NB = 4
LANES = 16
NH = D // (2 * LANES)
INV_K = 1.0 / K
ISLC = 128
UNROLL = 4


def _gcn_body(x_hbm, adj_hbm, table_hbm, tbf_hbm, out_hbm,
              x_v, edge_v, nb0, nb1, nb2, nb3, sf0, sf1, sf2, sf3,
              out0, out1, out2, out3,
              sem_e, sem_n0, sem_n1, sem_n2, sem_n3,
              sem_o0, sem_o1, sem_o2, sem_o3):
    wid = lax.axis_index("s") * NC + lax.axis_index("c")
    base = wid * BPW

    pltpu.sync_copy(x_hbm.at[pl.ds(base, BPW)], x_v)

    for j in range(BPW // ISLC):
        sl = pl.ds(j * ISLC, ISLC)
        pltpu.async_copy(adj_hbm.at[x_v.at[sl]], edge_v.at[sl], sem_e)
    for j in range(BPW // ISLC):
        sl = pl.ds(j * ISLC, ISLC)
        pltpu.make_async_copy(adj_hbm.at[x_v.at[sl]], edge_v.at[sl], sem_e).wait()

    def fire_nb(g, nb, sf, sem):
        for q in range(C):
            pltpu.async_copy(tbf_hbm.at[edge_v.at[g * C + q]], nb.at[q], sem)
        pltpu.async_copy(table_hbm.at[x_v.at[pl.ds(g * C, C)]], sf, sem)

    def drain_nb(g, nb, sf, sem):
        for q in range(C):
            pltpu.make_async_copy(
                tbf_hbm.at[edge_v.at[g * C + q]], nb.at[q], sem).wait()
        pltpu.make_async_copy(
            table_hbm.at[x_v.at[pl.ds(g * C, C)]], sf, sem).wait()

    def fire_out(g, out_v, sem):
        pltpu.async_copy(out_v, out_hbm.at[pl.ds(base + g * C, C)], sem)

    def drain_out(g, out_v, sem):
        pltpu.make_async_copy(
            out_v, out_hbm.at[pl.ds(base + g * C, C)], sem).wait()

    lane = lax.iota(jnp.int32, LANES)
    evens = [h * 2 * LANES + 2 * lane for h in range(NH)]
    odds = [e + 1 for e in evens]

    def compute(g, nb, sf, out_v):
        def qbody(q, carry):
            def red(k4, accs):
                new = list(accs)
                for dk in range(UNROLL):
                    row = UNROLL * k4 + dk
                    for h in range(NH):
                        a, b = plsc.unpack(
                            nb[q, row, pl.ds(h * 2 * LANES, 2 * LANES)],
                            format=plsc.PackFormat.INTERLEAVED,
                            preferred_element_type=jnp.float32)
                        new[2 * h] = new[2 * h] + a
                        new[2 * h + 1] = new[2 * h + 1] + b
                return tuple(new)

            zero = jnp.zeros((LANES,), jnp.float32)
            accs = lax.fori_loop(0, K // UNROLL, red, (zero,) * (2 * NH))
            qb = jnp.full((LANES,), q, jnp.int32)
            for h in range(NH):
                sa = plsc.load_gather(sf, [qb, evens[h]])
                sb = plsc.load_gather(sf, [qb, odds[h]])
                plsc.store_scatter(out_v, [qb, evens[h]],
                                   accs[2 * h] * INV_K + sa)
                plsc.store_scatter(out_v, [qb, odds[h]],
                                   accs[2 * h + 1] * INV_K + sb)
            return carry

        lax.fori_loop(0, C, qbody, 0)

    bufs = ((nb0, sf0, sem_n0, out0, sem_o0),
            (nb1, sf1, sem_n1, out1, sem_o1),
            (nb2, sf2, sem_n2, out2, sem_o2),
            (nb3, sf3, sem_n3, out3, sem_o3))
    for b, (nb, sf, semn, _, _o) in enumerate(bufs):
        fire_nb(b, nb, sf, semn)

    def step(i, carry):
        for b, (nb, sf, semn, out_v, semo) in enumerate(bufs):
            g = NB * i + b

            @pl.when(g >= NB)
            def _():
                drain_out(g - NB, out_v, semo)

            drain_nb(g, nb, sf, semn)
            compute(g, nb, sf, out_v)
            fire_out(g, out_v, semo)

            @pl.when(g + NB < NCH)
            def _():
                fire_nb(g + NB, nb, sf, semn)

        return carry

    lax.fori_loop(0, NCH // NB, step, 0)
    for b, (nb, sf, semn, out_v, semo) in enumerate(bufs):
        drain_out(NCH - NB + b, out_v, semo)


def kernel(X, adj, table):
    x = jnp.reshape(X, (B,)).astype(jnp.int32)
    adj32 = adj.astype(jnp.int32)
    tbf = table.astype(jnp.bfloat16)
    f = pl.kernel(
        _gcn_body,
        out_type=jax.ShapeDtypeStruct((B, D), jnp.float32),
        mesh=plsc.VectorSubcoreMesh(core_axis_name="c", subcore_axis_name="s"),
        compiler_params=pltpu.CompilerParams(
            use_tc_tiling_on_sc=False, needs_layout_passes=False),
        scratch_types=[
            pltpu.VMEM((BPW,), jnp.int32),
            pltpu.VMEM((BPW, K), jnp.int32),
            pltpu.VMEM((C, K, D), jnp.bfloat16),
            pltpu.VMEM((C, K, D), jnp.bfloat16),
            pltpu.VMEM((C, K, D), jnp.bfloat16),
            pltpu.VMEM((C, K, D), jnp.bfloat16),
            pltpu.VMEM((C, D), jnp.float32),
            pltpu.VMEM((C, D), jnp.float32),
            pltpu.VMEM((C, D), jnp.float32),
            pltpu.VMEM((C, D), jnp.float32),
            pltpu.VMEM((C, D), jnp.float32),
            pltpu.VMEM((C, D), jnp.float32),
            pltpu.VMEM((C, D), jnp.float32),
            pltpu.VMEM((C, D), jnp.float32),
            pltpu.SemaphoreType.DMA,
            pltpu.SemaphoreType.DMA,
            pltpu.SemaphoreType.DMA,
            pltpu.SemaphoreType.DMA,
            pltpu.SemaphoreType.DMA,
            pltpu.SemaphoreType.DMA,
            pltpu.SemaphoreType.DMA,
            pltpu.SemaphoreType.DMA,
            pltpu.SemaphoreType.DMA,
        ],
    )
    out = f(x, adj32, table, tbf)
    return jnp.reshape(out, (B, 1, D))


# speedup vs baseline: 1.4493x; 1.4493x over previous
import jax
import jax.numpy as jnp
from jax import lax
from jax.experimental import pallas as pl
from jax.experimental.pallas import tpu as pltpu
from jax.experimental.pallas import tpu_sc as plsc

N_NODES = 100000
K = 32
D = 128
B = 16384

NC = 2
NS = 16
NW = NC * NS
BPW = B // NW
C = 8
NCH = BPW // C
LANES = 16
NV = D // LANES
INV_K = 1.0 / K
ISLC = 128
UNROLL = 4


def _gcn_body(x_hbm, adj_hbm, table_hbm, out_hbm,
              x_v, edge_v, nb0, nb1, nb2, sf0, sf1, sf2, out0, out1, out2,
              sem_e, sem_n0, sem_n1, sem_n2, sem_o0, sem_o1, sem_o2):
    wid = lax.axis_index("s") * NC + lax.axis_index("c")
    base = wid * BPW

    pltpu.sync_copy(x_hbm.at[pl.ds(base, BPW)], x_v)

    for j in range(BPW // ISLC):
        sl = pl.ds(j * ISLC, ISLC)
        pltpu.async_copy(adj_hbm.at[x_v.at[sl]], edge_v.at[sl], sem_e)
    for j in range(BPW // ISLC):
        sl = pl.ds(j * ISLC, ISLC)
        pltpu.make_async_copy(adj_hbm.at[x_v.at[sl]], edge_v.at[sl], sem_e).wait()

    def fire_nb(g, nb, sf, sem):
        for q in range(C):
            pltpu.async_copy(table_hbm.at[edge_v.at[g * C + q]], nb.at[q], sem)
        pltpu.async_copy(table_hbm.at[x_v.at[pl.ds(g * C, C)]], sf, sem)

    def drain_nb(g, nb, sf, sem):
        for q in range(C):
            pltpu.make_async_copy(
                table_hbm.at[edge_v.at[g * C + q]], nb.at[q], sem).wait()
        pltpu.make_async_copy(
            table_hbm.at[x_v.at[pl.ds(g * C, C)]], sf, sem).wait()

    def fire_out(g, out_v, sem):
        pltpu.async_copy(out_v, out_hbm.at[pl.ds(base + g * C, C)], sem)

    def drain_out(g, out_v, sem):
        pltpu.make_async_copy(
            out_v, out_hbm.at[pl.ds(base + g * C, C)], sem).wait()

    def compute(g, nb, sf, out_v):
        def qbody(q, carry):
            def red(k4, accs):
                new = list(accs)
                for dk in range(UNROLL):
                    row = UNROLL * k4 + dk
                    for d in range(NV):
                        new[d] = new[d] + nb[q, row, pl.ds(d * LANES, LANES)]
                return tuple(new)

            zero = jnp.zeros((LANES,), jnp.float32)
            accs = lax.fori_loop(0, K // UNROLL, red, (zero,) * NV)
            for d in range(NV):
                dsl = pl.ds(d * LANES, LANES)
                out_v[q, dsl] = accs[d] * INV_K + sf[q, dsl]
            return carry

        lax.fori_loop(0, C, qbody, 0)

    NB = 3
    fire_nb(0, nb0, sf0, sem_n0)
    fire_nb(1, nb1, sf1, sem_n1)
    fire_nb(2, nb2, sf2, sem_n2)

    bufs = ((nb0, sf0, sem_n0, out0, sem_o0),
            (nb1, sf1, sem_n1, out1, sem_o1),
            (nb2, sf2, sem_n2, out2, sem_o2))

    def step(i, carry):
        for b, (nb, sf, semn, out_v, semo) in enumerate(bufs):
            g = NB * i + b

            @pl.when(g >= NB)
            def _():
                drain_out(g - NB, out_v, semo)

            drain_nb(g, nb, sf, semn)
            compute(g, nb, sf, out_v)
            fire_out(g, out_v, semo)

            @pl.when(g + NB < NCH)
            def _():
                fire_nb(g + NB, nb, sf, semn)

        return carry

    lax.fori_loop(0, NCH // NB, step, 0)
    g_last = (NCH // NB) * NB
    drain_out(g_last - NB, out0, sem_o0)
    drain_nb(g_last, nb0, sf0, sem_n0)
    compute(g_last, nb0, sf0, out0)
    fire_out(g_last, out0, sem_o0)
    drain_out(g_last - 2, out1, sem_o1)
    drain_out(g_last - 1, out2, sem_o2)
    drain_out(g_last, out0, sem_o0)


def kernel(X, adj, table):
    x = jnp.reshape(X, (B,)).astype(jnp.int32)
    adj32 = adj.astype(jnp.int32)
    f = pl.kernel(
        _gcn_body,
        out_type=jax.ShapeDtypeStruct((B, D), jnp.float32),
        mesh=plsc.VectorSubcoreMesh(core_axis_name="c", subcore_axis_name="s"),
        compiler_params=pltpu.CompilerParams(use_tc_tiling_on_sc=False),
        scratch_types=[
            pltpu.VMEM((BPW,), jnp.int32),
            pltpu.VMEM((BPW, K), jnp.int32),
            pltpu.VMEM((C, K, D), jnp.float32),
            pltpu.VMEM((C, K, D), jnp.float32),
            pltpu.VMEM((C, K, D), jnp.float32),
            pltpu.VMEM((C, D), jnp.float32),
            pltpu.VMEM((C, D), jnp.float32),
            pltpu.VMEM((C, D), jnp.float32),
            pltpu.VMEM((C, D), jnp.float32),
            pltpu.VMEM((C, D), jnp.float32),
            pltpu.VMEM((C, D), jnp.float32),
            pltpu.SemaphoreType.DMA,
            pltpu.SemaphoreType.DMA,
            pltpu.SemaphoreType.DMA,
            pltpu.SemaphoreType.DMA,
            pltpu.SemaphoreType.DMA,
            pltpu.SemaphoreType.DMA,
            pltpu.SemaphoreType.DMA,
        ],
    )
    out = f(x, adj32, table)
    return jnp.reshape(out, (B, 1, D))
